# baseline (speedup 1.0000x reference)
import functools

import jax
import jax.numpy as jnp
from jax.experimental import pallas as pl
from jax.experimental.pallas import tpu as pltpu

N = 10000
DIN = 128
DOUT = 128
S = 4
BM = 104
SLAB = N // S


def _h_kernel(x_ref, w_ref, h_ref):
    h_ref[...] = jnp.dot(x_ref[...], w_ref[...],
                         preferred_element_type=jnp.float32).astype(jnp.bfloat16)


def _main_kernel(a0_ref, a1_ref, a2_ref, a3_ref, h_ref, x_ref, w_own_ref,
                 bias_ref, out_ref):
    h = h_ref[...]
    w_own = w_own_ref[...]
    b = bias_ref[...]
    for s, a_ref in enumerate((a0_ref, a1_ref, a2_ref, a3_ref)):
        nbr = jnp.dot(a_ref[0].astype(jnp.bfloat16), h,
                      preferred_element_type=jnp.float32)
        own = jnp.dot(x_ref[s], w_own, preferred_element_type=jnp.float32)
        out_ref[s] = nbr + own + b


@functools.partial(jax.jit, static_argnames=())
def kernel(x, adj, weight_own, weight_nbr, bias):
    h = pl.pallas_call(
        _h_kernel,
        out_shape=jax.ShapeDtypeStruct((N, DOUT), jnp.bfloat16),
    )(x, weight_nbr)

    adj3 = adj.reshape(S, SLAB, N)
    x3 = x.reshape(S, SLAB, DIN)
    bias2d = bias.reshape(1, DOUT)
    grid = ((SLAB + BM - 1) // BM,)

    def adj_spec(s):
        return pl.BlockSpec((1, BM, N), lambda i, s=s: (s, i, 0))

    out = pl.pallas_call(
        _main_kernel,
        grid=grid,
        in_specs=[
            adj_spec(0), adj_spec(1), adj_spec(2), adj_spec(3),
            pl.BlockSpec((N, DOUT), lambda i: (0, 0)),
            pl.BlockSpec((S, BM, DIN), lambda i: (0, i, 0)),
            pl.BlockSpec((DIN, DOUT), lambda i: (0, 0)),
            pl.BlockSpec((1, DOUT), lambda i: (0, 0)),
        ],
        out_specs=pl.BlockSpec((S, BM, DOUT), lambda i: (0, i, 0)),
        out_shape=jax.ShapeDtypeStruct((S, SLAB, DOUT), jnp.float32),
        compiler_params=pltpu.CompilerParams(
            dimension_semantics=("arbitrary",),
        ),
    )(adj3, adj3, adj3, adj3, h, x3, weight_own, bias2d)
    return out.reshape(N, DOUT)


# speedup vs baseline: 3.0166x; 3.0166x over previous
import functools

import jax
import jax.numpy as jnp
from jax.experimental import pallas as pl
from jax.experimental.pallas import tpu as pltpu

N = 10000
DIN = 128
DOUT = 128
S = 5
BM = 40
SLAB = N // S


def _h_kernel(x_ref, w_ref, h_ref):
    h_ref[...] = jnp.dot(x_ref[...], w_ref[...],
                         preferred_element_type=jnp.float32).astype(jnp.bfloat16)


def _main_kernel(a0_ref, a1_ref, a2_ref, a3_ref, a4_ref, h_ref, x_ref, w_own_ref,
                 bias_ref, out_ref):
    h = h_ref[...]
    w_own = w_own_ref[...]
    b = bias_ref[...]
    for s, a_ref in enumerate((a0_ref, a1_ref, a2_ref, a3_ref, a4_ref)):
        nbr = jnp.dot(a_ref[0].astype(jnp.bfloat16), h,
                      preferred_element_type=jnp.float32)
        own = jnp.dot(x_ref[s], w_own, preferred_element_type=jnp.float32)
        out_ref[s] = nbr + own + b


@functools.partial(jax.jit, static_argnames=())
def kernel(x, adj, weight_own, weight_nbr, bias):
    h = pl.pallas_call(
        _h_kernel,
        out_shape=jax.ShapeDtypeStruct((N, DOUT), jnp.bfloat16),
    )(x, weight_nbr)

    adj3 = adj.reshape(S, SLAB, N)
    x3 = x.reshape(S, SLAB, DIN)
    bias2d = bias.reshape(1, DOUT)
    grid = ((SLAB + BM - 1) // BM,)

    def adj_spec(s):
        return pl.BlockSpec((1, BM, N), lambda i, s=s: (s, i, 0))

    out = pl.pallas_call(
        _main_kernel,
        grid=grid,
        in_specs=[
            adj_spec(0), adj_spec(1), adj_spec(2), adj_spec(3), adj_spec(4),
            pl.BlockSpec((N, DOUT), lambda i: (0, 0)),
            pl.BlockSpec((S, BM, DIN), lambda i: (0, i, 0)),
            pl.BlockSpec((DIN, DOUT), lambda i: (0, 0)),
            pl.BlockSpec((1, DOUT), lambda i: (0, 0)),
        ],
        out_specs=pl.BlockSpec((S, BM, DOUT), lambda i: (0, i, 0)),
        out_shape=jax.ShapeDtypeStruct((S, SLAB, DOUT), jnp.float32),
        compiler_params=pltpu.CompilerParams(
            dimension_semantics=("arbitrary",),
        ),
    )(adj3, adj3, adj3, adj3, adj3, h, x3, weight_own, bias2d)
    return out.reshape(N, DOUT)


# speedup vs baseline: 3.5410x; 1.1738x over previous
import functools

import jax
import jax.numpy as jnp
from jax.experimental import pallas as pl
from jax.experimental.pallas import tpu as pltpu

N = 10000
DIN = 128
DOUT = 128
BM = 200
HALF = N // 2


def _main_kernel(adj_t_ref, adj_b_ref, x_ref, w_own_ref, w_nbr_ref, bias_ref,
                 out_ref, h_ref):
    i = pl.program_id(0)

    @pl.when(i == 0)
    def _():
        h_ref[...] = jnp.dot(x_ref[...], w_nbr_ref[...],
                             preferred_element_type=jnp.float32
                             ).astype(jnp.bfloat16)

    h = h_ref[...]
    w_own = w_own_ref[...]
    b = bias_ref[...]
    x_t = x_ref[pl.ds(i * BM, BM), :]
    x_b = x_ref[pl.ds(HALF + i * BM, BM), :]
    top = jnp.dot(adj_t_ref[0].astype(jnp.bfloat16), h,
                  preferred_element_type=jnp.float32)
    bot = jnp.dot(adj_b_ref[0].astype(jnp.bfloat16), h,
                  preferred_element_type=jnp.float32)
    out_ref[0] = top + jnp.dot(x_t, w_own, preferred_element_type=jnp.float32) + b
    out_ref[1] = bot + jnp.dot(x_b, w_own, preferred_element_type=jnp.float32) + b


@functools.partial(jax.jit, static_argnames=())
def kernel(x, adj, weight_own, weight_nbr, bias):
    adj3 = adj.reshape(2, HALF, N)
    bias2d = bias.reshape(1, DOUT)
    grid = (HALF // BM,)
    out = pl.pallas_call(
        _main_kernel,
        grid=grid,
        in_specs=[
            pl.BlockSpec((1, BM, N), lambda i: (0, i, 0)),
            pl.BlockSpec((1, BM, N), lambda i: (1, i, 0)),
            pl.BlockSpec((N, DIN), lambda i: (0, 0)),
            pl.BlockSpec((DIN, DOUT), lambda i: (0, 0)),
            pl.BlockSpec((DIN, DOUT), lambda i: (0, 0)),
            pl.BlockSpec((1, DOUT), lambda i: (0, 0)),
        ],
        out_specs=pl.BlockSpec((2, BM, DOUT), lambda i: (0, i, 0)),
        out_shape=jax.ShapeDtypeStruct((2, HALF, DOUT), jnp.float32),
        scratch_shapes=[pltpu.VMEM((N, DOUT), jnp.bfloat16)],
        compiler_params=pltpu.CompilerParams(
            dimension_semantics=("arbitrary",),
        ),
    )(adj3, adj3, x, weight_own, weight_nbr, bias2d)
    return out.reshape(N, DOUT)
